# baseline (speedup 1.0000x reference)
import jax
import jax.numpy as jnp
from jax.experimental import pallas as pl

VOCAB = 1000
ROWS = 1024 * 50
BLOCK_ROWS = 512
NUM_BLOCKS = ROWS // BLOCK_ROWS


def _onehot_block(ids_ref, out_ref):
    ids = ids_ref[0, 0, :]
    iota = jax.lax.broadcasted_iota(jnp.int32, (BLOCK_ROWS, VOCAB), 1)
    out_ref[:, :] = (iota == ids[:, None]).astype(jnp.float32)


def kernel(input):
    ids = input.astype(jnp.int32).reshape(NUM_BLOCKS, 1, BLOCK_ROWS)
    out = pl.pallas_call(
        _onehot_block,
        grid=(NUM_BLOCKS,),
        in_specs=[pl.BlockSpec((1, 1, BLOCK_ROWS), lambda i: (i, 0, 0))],
        out_specs=pl.BlockSpec((BLOCK_ROWS, VOCAB), lambda i: (i, 0)),
        out_shape=jax.ShapeDtypeStruct((ROWS, VOCAB), jnp.float32),
    )(ids)
    return out.reshape(1024, 50, VOCAB)


# speedup vs baseline: 1.4355x; 1.4355x over previous
import jax
import jax.numpy as jnp
from jax.experimental import pallas as pl

VOCAB = 1000
BATCH = 1024
HIST = 50
BLOCK_B = 32


def _onehot_block(ids_ref, out_ref):
    ids = ids_ref[:, :]
    iota = jax.lax.broadcasted_iota(jnp.int32, (BLOCK_B, HIST, VOCAB), 2)
    out_ref[:, :, :] = (iota == ids[:, :, None]).astype(jnp.float32)


def kernel(input):
    ids = input.astype(jnp.int32)
    return pl.pallas_call(
        _onehot_block,
        grid=(BATCH // BLOCK_B,),
        in_specs=[pl.BlockSpec((BLOCK_B, HIST), lambda i: (i, 0))],
        out_specs=pl.BlockSpec((BLOCK_B, HIST, VOCAB), lambda i: (i, 0, 0)),
        out_shape=jax.ShapeDtypeStruct((BATCH, HIST, VOCAB), jnp.float32),
    )(ids)
